# baseline (speedup 1.0000x reference)
import functools

import jax
import jax.numpy as jnp
from jax import lax
from jax.experimental import pallas as pl
from jax.experimental.pallas import tpu as pltpu
from jax.experimental.pallas import tpu_sc as plsc

N = 10000
NP = 10240
E = 320000
D = 128

NC = 2
NS = 16
NW = NC * NS
CH = 64
NCHUNK = 160
EP = NW * NCHUNK * CH
EPW = EP // NW
NPAD_DST = 240
RPT = NP // NS
WCH = 64
WLOOPS = RPT // WCH

BR = 1280


def _make_seg_sum(with_deg):
    out_type = [jax.ShapeDtypeStruct((NC * NP, D), jnp.float32)]
    scratch = (
        [pltpu.VMEM((CH,), jnp.int32)] * 8 +
        [pltpu.VMEM((CH,), jnp.int32)] * 8 +
        [pltpu.VMEM((CH, D), jnp.float32)] * 4 +
        [pltpu.VMEM_SHARED((NP, D), jnp.float32)] +
        [pltpu.SemaphoreType.DMA] * 16
    )
    if with_deg:
        out_type.append(jax.ShapeDtypeStruct((NC * NP,), jnp.float32))
        scratch += [
            pltpu.VMEM((CH,), jnp.float32),
            pltpu.VMEM((RPT,), jnp.float32),
            pltpu.VMEM_SHARED((NP,), jnp.float32),
        ]

    def body(t_hbm, src_hbm, dst_hbm, out_hbm, *rest):
        if with_deg:
            deg_hbm = rest[0]
            rest = rest[1:]
        srcb = rest[0:8]
        dstb = rest[8:16]
        rowsb = rest[16:20]
        acc_sh = rest[20]
        isem = rest[21:29]
        gsem = rest[29:33]
        ssem = rest[33:37]
        if with_deg:
            ones_v, zdeg_v, deg_sh = rest[37:40]
        cid = lax.axis_index("c")
        sid = lax.axis_index("s")
        wid = sid * NC + cid
        row0 = sid * RPT

        def zb(i, c):
            for j in range(D // 16):
                rowsb[0][i, pl.ds(j * 16, 16)] = jnp.zeros((16,), jnp.float32)
            return c
        lax.fori_loop(0, WCH, zb, 0)
        for k in range(WLOOPS):
            pltpu.sync_copy(rowsb[0], acc_sh.at[pl.ds(row0 + k * WCH, WCH)])

        if with_deg:
            for q in range(CH // 16):
                ones_v[pl.ds(q * 16, 16)] = jnp.ones((16,), jnp.float32)
            for q in range(RPT // 16):
                zdeg_v[pl.ds(q * 16, 16)] = jnp.zeros((16,), jnp.float32)
            pltpu.sync_copy(zdeg_v, deg_sh.at[pl.ds(row0, RPT)])

        plsc.subcore_barrier()

        def idx_start(c, k):
            base = wid * EPW + c * CH
            pltpu.async_copy(src_hbm.at[pl.ds(base, CH)], srcb[k], isem[k])
            pltpu.async_copy(dst_hbm.at[pl.ds(base, CH)], dstb[k], isem[k])

        def idx_wait(c, k):
            base = wid * EPW + c * CH
            pltpu.make_async_copy(src_hbm.at[pl.ds(base, CH)], srcb[k], isem[k]).wait()
            pltpu.make_async_copy(dst_hbm.at[pl.ds(base, CH)], dstb[k], isem[k]).wait()

        def gather_start(k, b):
            pltpu.async_copy(t_hbm.at[srcb[k]], rowsb[b], gsem[b])

        def gather_wait(k, b):
            pltpu.make_async_copy(t_hbm.at[srcb[k]], rowsb[b], gsem[b]).wait()

        def scat_start(k, b):
            pltpu.async_copy(rowsb[b], acc_sh.at[dstb[k]], ssem[b], add=True)
            if with_deg:
                pltpu.sync_copy(ones_v, deg_sh.at[dstb[k]], add=True)

        def scat_wait(k, b):
            pltpu.make_async_copy(rowsb[b], acc_sh.at[dstb[k]], ssem[b]).wait()

        for c0 in range(6):
            idx_start(c0, c0)
        idx_wait(0, 0)
        gather_start(0, 0)
        idx_wait(1, 1)
        gather_start(1, 1)
        gather_wait(0, 0)
        scat_start(0, 0)
        idx_wait(2, 2)
        gather_start(2, 2)
        gather_wait(1, 1)
        scat_start(1, 1)
        idx_wait(3, 3)
        gather_start(3, 3)
        gather_wait(2, 2)
        scat_start(2, 2)

        def steady(g, carry):
            for p in range(8):
                c = 4 + 8 * g + p
                k = (4 + p) % 8
                b = (4 + p) % 4
                scat_wait(k, b)
                idx_wait(c, k)
                gather_start(k, b)
                idx_start(c + 2, (4 + p + 2) % 8)
                gather_wait((k + 7) % 8, (b + 3) % 4)
                scat_start((k + 7) % 8, (b + 3) % 4)
            return carry
        lax.fori_loop(0, (NCHUNK - 8) // 8, steady, 0)

        for c in range(NCHUNK - 4, NCHUNK):
            k = c % 8
            b = c % 4
            scat_wait(k, b)
            idx_wait(c, k)
            gather_start(k, b)
            if c + 2 < NCHUNK:
                idx_start(c + 2, (c + 2) % 8)
            gather_wait((c - 1) % 8, (c - 1) % 4)
            scat_start((c - 1) % 8, (c - 1) % 4)
        gather_wait((NCHUNK - 1) % 8, (NCHUNK - 1) % 4)
        scat_start((NCHUNK - 1) % 8, (NCHUNK - 1) % 4)
        for c in range(NCHUNK - 4, NCHUNK):
            scat_wait(c % 8, c % 4)

        plsc.subcore_barrier()

        for k in range(WLOOPS):
            r = row0 + k * WCH
            pltpu.sync_copy(acc_sh.at[pl.ds(r, WCH)], rowsb[0])
            pltpu.sync_copy(rowsb[0], out_hbm.at[pl.ds(cid * NP + r, WCH)])
        if with_deg:
            pltpu.sync_copy(deg_sh.at[pl.ds(row0, RPT)], zdeg_v)
            pltpu.sync_copy(zdeg_v, deg_hbm.at[pl.ds(cid * NP + row0, RPT)])

    mesh = plsc.VectorSubcoreMesh(
        core_axis_name="c", subcore_axis_name="s",
        num_cores=NC, num_subcores=NS)
    return pl.kernel(
        body,
        out_type=tuple(out_type) if with_deg else out_type[0],
        mesh=mesh,
        scratch_types=scratch,
    )


@functools.lru_cache(maxsize=None)
def _seg_sum(with_deg):
    return _make_seg_sum(with_deg)


def _pre_body(x_ref, wl_ref, wr_ref, b_ref, y_ref, z_ref):
    x = x_ref[...]
    y_ref[...] = jnp.dot(x, wl_ref[...], preferred_element_type=jnp.float32)
    z_ref[...] = jnp.dot(x, wr_ref[...], preferred_element_type=jnp.float32) + b_ref[...]


_pre = pl.pallas_call(
    _pre_body,
    grid=(NP // BR,),
    in_specs=[
        pl.BlockSpec((BR, D), lambda i: (i, 0)),
        pl.BlockSpec((D, D), lambda i: (0, 0)),
        pl.BlockSpec((D, D), lambda i: (0, 0)),
        pl.BlockSpec((1, D), lambda i: (0, 0)),
    ],
    out_specs=[
        pl.BlockSpec((BR, D), lambda i: (i, 0)),
        pl.BlockSpec((BR, D), lambda i: (i, 0)),
    ],
    out_shape=[jax.ShapeDtypeStruct((NP, D), jnp.float32)] * 2,
)


def _mid_body(p_ref, d_ref, z1_ref, wl_ref, wr_ref, b_ref, y_ref, z_ref):
    s = p_ref[0] + p_ref[1]
    d = d_ref[...]
    deg = d[0] + d[1]
    rcp = 1.0 / jnp.maximum(deg, 1.0)
    h = jnp.maximum(s * rcp + z1_ref[...], 0.0)
    y_ref[...] = jnp.dot(h, wl_ref[...], preferred_element_type=jnp.float32)
    z_ref[...] = jnp.dot(h, wr_ref[...], preferred_element_type=jnp.float32) + b_ref[...]


_mid = pl.pallas_call(
    _mid_body,
    grid=(NP // BR,),
    in_specs=[
        pl.BlockSpec((NC, BR, D), lambda i: (0, i, 0)),
        pl.BlockSpec((NC, BR, 1), lambda i: (0, i, 0)),
        pl.BlockSpec((BR, D), lambda i: (i, 0)),
        pl.BlockSpec((D, D), lambda i: (0, 0)),
        pl.BlockSpec((D, D), lambda i: (0, 0)),
        pl.BlockSpec((1, D), lambda i: (0, 0)),
    ],
    out_specs=[
        pl.BlockSpec((BR, D), lambda i: (i, 0)),
        pl.BlockSpec((BR, D), lambda i: (i, 0)),
    ],
    out_shape=[jax.ShapeDtypeStruct((NP, D), jnp.float32)] * 2,
)


def _post_body(p_ref, d_ref, z2_ref, o_ref):
    s = p_ref[0] + p_ref[1]
    d = d_ref[...]
    deg = d[0] + d[1]
    rcp = 1.0 / jnp.maximum(deg, 1.0)
    o_ref[...] = s * rcp + z2_ref[...]


_post = pl.pallas_call(
    _post_body,
    grid=(NP // BR,),
    in_specs=[
        pl.BlockSpec((NC, BR, D), lambda i: (0, i, 0)),
        pl.BlockSpec((NC, BR, 1), lambda i: (0, i, 0)),
        pl.BlockSpec((BR, D), lambda i: (i, 0)),
    ],
    out_specs=pl.BlockSpec((BR, D), lambda i: (i, 0)),
    out_shape=jax.ShapeDtypeStruct((NP, D), jnp.float32),
)


def kernel(x, edge_index, W1_l, W1_r, b1, W2_l, W2_r, b2):
    npad = EP - E
    src = jnp.concatenate(
        [edge_index[0].astype(jnp.int32),
         jnp.arange(npad, dtype=jnp.int32) % N])
    dst = jnp.concatenate(
        [edge_index[1].astype(jnp.int32),
         N + (jnp.arange(npad, dtype=jnp.int32) % NPAD_DST)])
    xp = jnp.pad(x, ((0, NP - N), (0, 0)))
    y1, z1 = _pre(xp, W1_l, W1_r, b1.reshape(1, D))
    p1, d1 = _seg_sum(True)(y1, src, dst)
    p1 = p1.reshape(NC, NP, D)
    d1 = d1.reshape(NC, NP, 1)
    y2, z2 = _mid(p1, d1, z1, W2_l, W2_r, b2.reshape(1, D))
    p2 = _seg_sum(False)(y2, src, dst).reshape(NC, NP, D)
    return _post(p2, d1, z2)[:N]


# speedup vs baseline: 1.0699x; 1.0699x over previous
import functools

import jax
import jax.numpy as jnp
from jax import lax
from jax.experimental import pallas as pl
from jax.experimental.pallas import tpu as pltpu
from jax.experimental.pallas import tpu_sc as plsc

N = 10000
NP = 10240
E = 320000
D = 128

NC = 2
NS = 16
NW = NC * NS
CH = 128
NCHUNK = 80
EP = NW * NCHUNK * CH
EPW = EP // NW
NPAD_DST = 240
RPT = NP // NS
WCH = 128
WLOOPS = RPT // WCH

BR = 1280


def _make_seg_sum(with_deg):
    out_type = [jax.ShapeDtypeStruct((NC * NP, D), jnp.float32)]
    scratch = (
        [pltpu.VMEM((CH,), jnp.int32)] * 4 +
        [pltpu.VMEM((CH,), jnp.int32)] * 4 +
        [pltpu.VMEM((CH, D), jnp.float32)] * 2 +
        [pltpu.VMEM_SHARED((NP, D), jnp.float32)] +
        [pltpu.SemaphoreType.DMA] * 8
    )
    if with_deg:
        out_type.append(jax.ShapeDtypeStruct((NC * NP,), jnp.float32))
        scratch += [
            pltpu.VMEM((CH,), jnp.float32),
            pltpu.VMEM((RPT,), jnp.float32),
            pltpu.VMEM_SHARED((NP,), jnp.float32),
        ]

    def body(t_hbm, src_hbm, dst_hbm, out_hbm, *rest):
        if with_deg:
            deg_hbm = rest[0]
            rest = rest[1:]
        srcb = rest[0:4]
        dstb = rest[4:8]
        rowsb = rest[8:10]
        acc_sh = rest[10]
        isem = rest[11:15]
        gsem = rest[15:17]
        ssem = rest[17:19]
        if with_deg:
            ones_v, zdeg_v, deg_sh = rest[19:22]
        cid = lax.axis_index("c")
        sid = lax.axis_index("s")
        wid = sid * NC + cid
        row0 = sid * RPT

        def zb(i, c):
            for j in range(D // 16):
                rowsb[0][i, pl.ds(j * 16, 16)] = jnp.zeros((16,), jnp.float32)
            return c
        lax.fori_loop(0, WCH, zb, 0)
        for k in range(WLOOPS):
            pltpu.sync_copy(rowsb[0], acc_sh.at[pl.ds(row0 + k * WCH, WCH)])

        if with_deg:
            for q in range(CH // 16):
                ones_v[pl.ds(q * 16, 16)] = jnp.ones((16,), jnp.float32)
            for q in range(RPT // 16):
                zdeg_v[pl.ds(q * 16, 16)] = jnp.zeros((16,), jnp.float32)
            pltpu.sync_copy(zdeg_v, deg_sh.at[pl.ds(row0, RPT)])

        plsc.subcore_barrier()

        def idx_start(c, k):
            base = wid * EPW + c * CH
            pltpu.async_copy(src_hbm.at[pl.ds(base, CH)], srcb[k], isem[k])
            pltpu.async_copy(dst_hbm.at[pl.ds(base, CH)], dstb[k], isem[k])

        def idx_wait(c, k):
            base = wid * EPW + c * CH
            pltpu.make_async_copy(src_hbm.at[pl.ds(base, CH)], srcb[k], isem[k]).wait()
            pltpu.make_async_copy(dst_hbm.at[pl.ds(base, CH)], dstb[k], isem[k]).wait()

        def gather_start(k, b):
            pltpu.async_copy(t_hbm.at[srcb[k]], rowsb[b], gsem[b])

        def gather_wait(k, b):
            pltpu.make_async_copy(t_hbm.at[srcb[k]], rowsb[b], gsem[b]).wait()

        def scat_start(k, b):
            pltpu.async_copy(rowsb[b], acc_sh.at[dstb[k]], ssem[b], add=True)
            if with_deg:
                pltpu.sync_copy(ones_v, deg_sh.at[dstb[k]], add=True)

        def scat_wait(k, b):
            pltpu.make_async_copy(rowsb[b], acc_sh.at[dstb[k]], ssem[b]).wait()

        idx_start(0, 0)
        idx_start(1, 1)
        idx_start(2, 2)
        idx_start(3, 3)
        idx_wait(0, 0)
        gather_start(0, 0)
        idx_wait(1, 1)
        gather_start(1, 1)
        gather_wait(0, 0)
        scat_start(0, 0)

        def steady(g, carry):
            for p in range(4):
                c = 2 + 4 * g + p
                k = (2 + p) % 4
                b = p % 2
                scat_wait(k, b)
                idx_wait(c, k)
                gather_start(k, b)
                idx_start(c + 2, p)
                gather_wait((k + 3) % 4, 1 - b)
                scat_start((k + 3) % 4, 1 - b)
            return carry
        lax.fori_loop(0, (NCHUNK - 4) // 4, steady, 0)

        scat_wait(2, 0)
        idx_wait(NCHUNK - 2, 2)
        gather_start(2, 0)
        gather_wait(1, 1)
        scat_start(1, 1)
        scat_wait(3, 1)
        idx_wait(NCHUNK - 1, 3)
        gather_start(3, 1)
        gather_wait(2, 0)
        scat_start(2, 0)
        gather_wait(3, 1)
        scat_start(3, 1)
        scat_wait(2, 0)
        scat_wait(3, 1)

        plsc.subcore_barrier()

        for k in range(WLOOPS):
            r = row0 + k * WCH
            pltpu.sync_copy(acc_sh.at[pl.ds(r, WCH)], rowsb[0])
            pltpu.sync_copy(rowsb[0], out_hbm.at[pl.ds(cid * NP + r, WCH)])
        if with_deg:
            pltpu.sync_copy(deg_sh.at[pl.ds(row0, RPT)], zdeg_v)
            pltpu.sync_copy(zdeg_v, deg_hbm.at[pl.ds(cid * NP + row0, RPT)])

    mesh = plsc.VectorSubcoreMesh(
        core_axis_name="c", subcore_axis_name="s",
        num_cores=NC, num_subcores=NS)
    return pl.kernel(
        body,
        out_type=tuple(out_type) if with_deg else out_type[0],
        mesh=mesh,
        scratch_types=scratch,
    )


@functools.lru_cache(maxsize=None)
def _seg_sum(with_deg):
    return _make_seg_sum(with_deg)


def _pre_body(x_ref, wl_ref, wr_ref, b_ref, y_ref, z_ref):
    x = x_ref[...]
    y_ref[...] = jnp.dot(x, wl_ref[...], preferred_element_type=jnp.float32)
    z_ref[...] = jnp.dot(x, wr_ref[...], preferred_element_type=jnp.float32) + b_ref[...]


_pre = pl.pallas_call(
    _pre_body,
    grid=(NP // BR,),
    in_specs=[
        pl.BlockSpec((BR, D), lambda i: (i, 0)),
        pl.BlockSpec((D, D), lambda i: (0, 0)),
        pl.BlockSpec((D, D), lambda i: (0, 0)),
        pl.BlockSpec((1, D), lambda i: (0, 0)),
    ],
    out_specs=[
        pl.BlockSpec((BR, D), lambda i: (i, 0)),
        pl.BlockSpec((BR, D), lambda i: (i, 0)),
    ],
    out_shape=[jax.ShapeDtypeStruct((NP, D), jnp.float32)] * 2,
)


def _mid_body(p_ref, d_ref, z1_ref, wl_ref, wr_ref, b_ref, y_ref, z_ref):
    s = p_ref[0] + p_ref[1]
    d = d_ref[...]
    deg = d[0] + d[1]
    rcp = 1.0 / jnp.maximum(deg, 1.0)
    h = jnp.maximum(s * rcp + z1_ref[...], 0.0)
    y_ref[...] = jnp.dot(h, wl_ref[...], preferred_element_type=jnp.float32)
    z_ref[...] = jnp.dot(h, wr_ref[...], preferred_element_type=jnp.float32) + b_ref[...]


_mid = pl.pallas_call(
    _mid_body,
    grid=(NP // BR,),
    in_specs=[
        pl.BlockSpec((NC, BR, D), lambda i: (0, i, 0)),
        pl.BlockSpec((NC, BR, 1), lambda i: (0, i, 0)),
        pl.BlockSpec((BR, D), lambda i: (i, 0)),
        pl.BlockSpec((D, D), lambda i: (0, 0)),
        pl.BlockSpec((D, D), lambda i: (0, 0)),
        pl.BlockSpec((1, D), lambda i: (0, 0)),
    ],
    out_specs=[
        pl.BlockSpec((BR, D), lambda i: (i, 0)),
        pl.BlockSpec((BR, D), lambda i: (i, 0)),
    ],
    out_shape=[jax.ShapeDtypeStruct((NP, D), jnp.float32)] * 2,
)


def _post_body(p_ref, d_ref, z2_ref, o_ref):
    s = p_ref[0] + p_ref[1]
    d = d_ref[...]
    deg = d[0] + d[1]
    rcp = 1.0 / jnp.maximum(deg, 1.0)
    o_ref[...] = s * rcp + z2_ref[...]


_post = pl.pallas_call(
    _post_body,
    grid=(NP // BR,),
    in_specs=[
        pl.BlockSpec((NC, BR, D), lambda i: (0, i, 0)),
        pl.BlockSpec((NC, BR, 1), lambda i: (0, i, 0)),
        pl.BlockSpec((BR, D), lambda i: (i, 0)),
    ],
    out_specs=pl.BlockSpec((BR, D), lambda i: (i, 0)),
    out_shape=jax.ShapeDtypeStruct((NP, D), jnp.float32),
)


def kernel(x, edge_index, W1_l, W1_r, b1, W2_l, W2_r, b2):
    npad = EP - E
    src = jnp.concatenate(
        [edge_index[0].astype(jnp.int32),
         jnp.arange(npad, dtype=jnp.int32) % N])
    dst = jnp.concatenate(
        [edge_index[1].astype(jnp.int32),
         N + (jnp.arange(npad, dtype=jnp.int32) % NPAD_DST)])
    xp = jnp.pad(x, ((0, NP - N), (0, 0)))
    y1, z1 = _pre(xp, W1_l, W1_r, b1.reshape(1, D))
    p1, d1 = _seg_sum(True)(y1, src, dst)
    p1 = p1.reshape(NC, NP, D)
    d1 = d1.reshape(NC, NP, 1)
    y2, z2 = _mid(p1, d1, z1, W2_l, W2_r, b2.reshape(1, D))
    p2 = _seg_sum(False)(y2, src, dst).reshape(NC, NP, D)
    return _post(p2, d1, z2)[:N]
